# Initial kernel scaffold; baseline (speedup 1.0000x reference)
import jax
import jax.numpy as jnp
from jax.experimental import pallas as pl


def kernel(x, params, edge_index, batch):
    raise NotImplementedError("write your pallas kernel here")


# speedup vs baseline: 3.9108x; 3.9108x over previous
import functools

import jax
import jax.numpy as jnp
from jax import lax
from jax.experimental import pallas as pl
from jax.experimental.pallas import tpu as pltpu
from jax.experimental.pallas import tpu_sc as plsc

_N = 10000
_E = 320000
_D = 128
_G = 128
_LAYERS = 4

_NC = 2
_NS = 16
_NW = _NC * _NS

_EPT = _E // _NS
_CH = 80
_NCH = _EPT // _CH
_HALF = _N // _NC
_NDUMP = 64
_AROWS = _HALF + _NDUMP
_ZR = 312
_ZTAIL = _AROWS - _NS * _ZR
_WTAIL = _HALF - _NS * _ZR

_EPW = _E // _NW
_CHD = 125
_NCHD = _EPW // _CHD
_RPT = 624
_TAIL = _N - _NS * _RPT
_TAILOFF = _NS * _RPT

_BLK = 1000
_NBLK = _N // _BLK


def _sc_prep_body(dst20_hbm, dre_hbm, dst20_v, dre_v):
    c = lax.axis_index("c")
    s = lax.axis_index("s")
    pltpu.sync_copy(dst20_hbm.at[s], dst20_v)
    base = c * _HALF

    def rbody(j, carry):
        for k in range(_CH // 16):
            d = dst20_v[j, pl.ds(k * 16, 16)]
            idx = d - base
            ok = (idx >= 0) & (idx < _HALF)
            dre_v[j, pl.ds(k * 16, 16)] = jnp.where(
                ok, idx, _HALF + (d & (_NDUMP - 1)))
        return carry

    lax.fori_loop(0, _NCH, rbody, 0)
    pltpu.sync_copy(dre_v, dre_hbm.at[c, s])


def _sc_agg_body(h_hbm, src20_hbm, dre_hbm, zrows_hbm, out_hbm,
                 src_v, dst_v, buf_v, sem0, sem1, agg_sh):
    c = lax.axis_index("c")
    s = lax.axis_index("s")
    pltpu.sync_copy(src20_hbm.at[s], src_v)
    pltpu.sync_copy(dre_hbm.at[c, s], dst_v)
    pltpu.sync_copy(zrows_hbm, agg_sh.at[pl.ds(s * _ZR, _ZR)])

    @pl.when(s == 0)
    def _():
        pltpu.sync_copy(zrows_hbm.at[pl.ds(0, _ZTAIL)],
                        agg_sh.at[pl.ds(_NS * _ZR, _ZTAIL)])

    plsc.subcore_barrier()

    def body(j, carry):
        cp0 = pltpu.async_copy(h_hbm.at[src_v.at[2 * j]], buf_v.at[0], sem0)
        cp1 = pltpu.async_copy(h_hbm.at[src_v.at[2 * j + 1]], buf_v.at[1], sem1)
        cp0.wait()
        pltpu.sync_copy(buf_v.at[0], agg_sh.at[dst_v.at[2 * j]], add=True)
        cp1.wait()
        pltpu.sync_copy(buf_v.at[1], agg_sh.at[dst_v.at[2 * j + 1]], add=True)
        return carry

    lax.fori_loop(0, _NCH // 2, body, 0)
    plsc.subcore_barrier()
    pltpu.sync_copy(agg_sh.at[pl.ds(s * _ZR, _ZR)],
                    out_hbm.at[c, pl.ds(s * _ZR, _ZR)])

    @pl.when(s == 0)
    def _():
        pltpu.sync_copy(agg_sh.at[pl.ds(_NS * _ZR, _WTAIL)],
                        out_hbm.at[c, pl.ds(_NS * _ZR, _WTAIL)])


@functools.cache
def _sc_kernels():
    sc_mesh = plsc.VectorSubcoreMesh(
        core_axis_name="c", subcore_axis_name="s",
        num_cores=_NC, num_subcores=_NS)
    sc_prep = pl.kernel(
        _sc_prep_body,
        out_type=jax.ShapeDtypeStruct((_NC, _NS, _NCH, _CH), jnp.int32),
        mesh=sc_mesh,
        scratch_types=[
            pltpu.VMEM((_NCH, _CH), jnp.int32),
            pltpu.VMEM((_NCH, _CH), jnp.int32),
        ],
    )
    sc_agg = pl.kernel(
        _sc_agg_body,
        out_type=jax.ShapeDtypeStruct((_NC, _HALF, _D), jnp.float32),
        mesh=sc_mesh,
        scratch_types=[
            pltpu.VMEM((_NCH, _CH), jnp.int32),
            pltpu.VMEM((_NCH, _CH), jnp.int32),
            pltpu.VMEM((2, _CH, _D), jnp.float32),
            pltpu.SemaphoreType.DMA,
            pltpu.SemaphoreType.DMA,
            pltpu.VMEM_SHARED((_AROWS, _D), jnp.float32),
        ],
    )
    return sc_prep, sc_agg


def _tc_prep_body(batch_ref, cnt_ref, pnt_ref, deginv_ref):
    b = batch_ref[...]
    gids = lax.broadcasted_iota(jnp.int32, (_N, _G), 1)
    p = (b == gids).astype(jnp.float32)
    gcnt = jnp.sum(p, axis=0, keepdims=True)
    pnt_ref[...] = p / jnp.maximum(gcnt, 1.0)
    deg = jnp.concatenate([cnt_ref[0, :, :1], cnt_ref[1, :, :1]], axis=0)
    deginv_ref[...] = 1.0 / jnp.maximum(deg, 1.0)


_tc_prep = pl.pallas_call(
    _tc_prep_body,
    out_shape=(
        jax.ShapeDtypeStruct((_N, _G), jnp.float32),
        jax.ShapeDtypeStruct((_N, 1), jnp.float32),
    ),
)


def _tc_a_body(h_ref, parts_ref, deginv_ref, w1_ref, b1_ref, w2_ref, b2_ref,
               z_ref, stats_ref):
    i = pl.program_id(0)
    agg = parts_ref[0] * deginv_ref[...]
    zin = h_ref[...] + agg
    t = jnp.dot(zin, w1_ref[...], preferred_element_type=jnp.float32)
    t = jnp.maximum(t + b1_ref[...], 0.0)
    z = jnp.dot(t, w2_ref[...], preferred_element_type=jnp.float32)
    z = jnp.maximum(z + b2_ref[...], 0.0)
    z_ref[...] = z
    st = jnp.concatenate(
        [jnp.sum(z, axis=0, keepdims=True),
         jnp.sum(z * z, axis=0, keepdims=True),
         jnp.zeros((6, _D), jnp.float32)], axis=0)

    @pl.when(i == 0)
    def _():
        stats_ref[...] = st

    @pl.when(i != 0)
    def _():
        stats_ref[...] += st


_tc_a = pl.pallas_call(
    _tc_a_body,
    grid=(_NBLK,),
    in_specs=[
        pl.BlockSpec((_BLK, _D), lambda i: (i, 0)),
        pl.BlockSpec((1, _BLK, _D),
                     lambda i: (i // (_NBLK // _NC), i % (_NBLK // _NC), 0)),
        pl.BlockSpec((_BLK, 1), lambda i: (i, 0)),
        pl.BlockSpec((_D, _D), lambda i: (0, 0)),
        pl.BlockSpec((1, _D), lambda i: (0, 0)),
        pl.BlockSpec((_D, _D), lambda i: (0, 0)),
        pl.BlockSpec((1, _D), lambda i: (0, 0)),
    ],
    out_specs=(
        pl.BlockSpec((_BLK, _D), lambda i: (i, 0)),
        pl.BlockSpec((8, _D), lambda i: (0, 0)),
    ),
    out_shape=(
        jax.ShapeDtypeStruct((_N, _D), jnp.float32),
        jax.ShapeDtypeStruct((8, _D), jnp.float32),
    ),
)


def _tc_b_body(z_ref, stats_ref, g_ref, be_ref, npin_ref, pnt_ref, gpin_ref,
               h_ref, npout_ref, gpout_ref):
    i = pl.program_id(0)
    mu = stats_ref[0:1, :] / _N
    var = stats_ref[1:2, :] / _N - mu * mu
    inv = lax.rsqrt(var + 1e-5)
    hb = (z_ref[...] - mu) * (inv * g_ref[...]) + be_ref[...]
    h_ref[...] = hb
    npout_ref[...] = npin_ref[...] + hb
    contrib = lax.dot_general(pnt_ref[...], hb, (((0,), (0,)), ((), ())),
                              preferred_element_type=jnp.float32)

    @pl.when(i == 0)
    def _():
        gpout_ref[...] = gpin_ref[...] + contrib

    @pl.when(i != 0)
    def _():
        gpout_ref[...] += contrib


_tc_b = pl.pallas_call(
    _tc_b_body,
    grid=(_NBLK,),
    in_specs=[
        pl.BlockSpec((_BLK, _D), lambda i: (i, 0)),
        pl.BlockSpec((8, _D), lambda i: (0, 0)),
        pl.BlockSpec((1, _D), lambda i: (0, 0)),
        pl.BlockSpec((1, _D), lambda i: (0, 0)),
        pl.BlockSpec((_BLK, _D), lambda i: (i, 0)),
        pl.BlockSpec((_BLK, _G), lambda i: (i, 0)),
        pl.BlockSpec((_G, _D), lambda i: (0, 0)),
    ],
    out_specs=(
        pl.BlockSpec((_BLK, _D), lambda i: (i, 0)),
        pl.BlockSpec((_BLK, _D), lambda i: (i, 0)),
        pl.BlockSpec((_G, _D), lambda i: (0, 0)),
    ),
    out_shape=(
        jax.ShapeDtypeStruct((_N, _D), jnp.float32),
        jax.ShapeDtypeStruct((_N, _D), jnp.float32),
        jax.ShapeDtypeStruct((_G, _D), jnp.float32),
    ),
)


def kernel(x, params, edge_index, batch):
    src20 = edge_index[0].reshape(_NS, _NCH, _CH)
    dst20 = edge_index[1].reshape(_NS, _NCH, _CH)
    zrows = jnp.zeros((_ZR, _D), jnp.float32)
    ones_nd = jnp.ones((_N, _D), jnp.float32)
    batch2 = batch.reshape(_N, 1)

    sc_prep, sc_agg = _sc_kernels()
    dre = sc_prep(dst20)
    cnt = sc_agg(ones_nd, src20, dre, zrows)
    pnt, deginv = _tc_prep(batch2, cnt)

    h = x
    npool = jnp.zeros((_N, _D), jnp.float32)
    gpool = jnp.zeros((_G, _D), jnp.float32)
    for l in range(_LAYERS):
        parts = sc_agg(h, src20, dre, zrows)
        z, stats = _tc_a(h, parts, deginv,
                         params['W1_%d' % l], params['b1_%d' % l].reshape(1, _D),
                         params['W2_%d' % l], params['b2_%d' % l].reshape(1, _D))
        h, npool, gpool = _tc_b(z, stats,
                                params['g_%d' % l].reshape(1, _D),
                                params['be_%d' % l].reshape(1, _D),
                                npool, pnt, gpool)
    return (npool, gpool)


# speedup vs baseline: 4.4731x; 1.1438x over previous
import functools

import jax
import jax.numpy as jnp
from jax import lax
from jax.experimental import pallas as pl
from jax.experimental.pallas import tpu as pltpu
from jax.experimental.pallas import tpu_sc as plsc

_N = 10000
_E = 320000
_D = 128
_G = 128
_LAYERS = 4

_NC = 2
_NS = 16
_NW = _NC * _NS

_EPT = _E // _NS
_CH = 80
_NCH = _EPT // _CH
_NCH2 = _NCH
_CAP = _NCH2 * _CH
_HLF1 = 128
_HALF = _N // _NC
_NDUMP = 64
_AROWS = _HALF + _NDUMP
_ZR = 312
_ZTAIL = _AROWS - _NS * _ZR
_WTAIL = _HALF - _NS * _ZR

_BLK = 1000
_NBLK = _N // _BLK


def _sc_prep_body(src20_hbm, dst20_hbm, csrc_hbm, cdst_hbm, nlp_hbm,
                  src_v, dst_v, csrc_v, cdst_v, nlp_v):
    c = lax.axis_index("c")
    s = lax.axis_index("s")
    base = c * _HALF

    nlp_v[...] = jnp.zeros((16,), jnp.int32)

    def gbody(j, carry):
        for k in range(_CH // 16):
            d = dst_v[j, pl.ds(k * 16, 16)]
            sv = src_v[j, pl.ds(k * 16, 16)]
            cntv = nlp_v[...]
            idx = d - base
            ok = (idx >= 0) & (idx < _HALF)
            cs = plsc.cumsum(jnp.where(ok, 1, 0))
            pos = jnp.maximum(cs + cntv - 1, 0)
            row = lax.div(pos, jnp.full((16,), _CH, jnp.int32))
            col = pos - row * _CH
            plsc.store_scatter(csrc_v, [row, col], sv, mask=ok)
            plsc.store_scatter(cdst_v, [row, col], idx, mask=ok)
            nlp_v[...] = cntv + plsc.all_reduce_population_count(ok)
        return carry

    for off, n in ((0, _HLF1), (_HLF1, _NCH - _HLF1)):
        pltpu.sync_copy(src20_hbm.at[s, pl.ds(off, n)], src_v.at[pl.ds(0, n)])
        pltpu.sync_copy(dst20_hbm.at[s, pl.ds(off, n)], dst_v.at[pl.ds(0, n)])
        lax.fori_loop(0, n, gbody, 0)
    cntv = nlp_v[...]

    lanes = lax.iota(jnp.int32, 16)
    zsrc = jnp.zeros((16,), jnp.int32)
    chv = jnp.full((16,), _CH, jnp.int32)
    nloopsv = lax.div(cntv + 2 * _CH - 1, jnp.full((16,), 2 * _CH, jnp.int32))
    roundv = nloopsv * (2 * _CH)
    for p in range(10):
        pos = cntv + p * 16 + lanes
        okp = pos < roundv
        row = lax.div(jnp.minimum(pos, _CAP - 1), chv)
        col = jnp.minimum(pos, _CAP - 1) - row * _CH
        plsc.store_scatter(csrc_v, [row, col], zsrc, mask=okp)
        plsc.store_scatter(cdst_v, [row, col],
                           _HALF + (pos & (_NDUMP - 1)), mask=okp)

    nlp_v[...] = nloopsv
    pltpu.sync_copy(csrc_v, csrc_hbm.at[c, s])
    pltpu.sync_copy(cdst_v, cdst_hbm.at[c, s])
    pltpu.sync_copy(nlp_v, nlp_hbm.at[c, s])


def _sc_agg_body(h_hbm, csrc_hbm, cdst_hbm, nlp_hbm, zrows_hbm, out_hbm,
                 src_v, dst_v, nlp_v, buf_v, sem0, sem1, agg_sh):
    c = lax.axis_index("c")
    s = lax.axis_index("s")
    pltpu.sync_copy(csrc_hbm.at[c, s], src_v)
    pltpu.sync_copy(cdst_hbm.at[c, s], dst_v)
    pltpu.sync_copy(nlp_hbm.at[c, s], nlp_v)
    pltpu.sync_copy(zrows_hbm, agg_sh.at[pl.ds(s * _ZR, _ZR)])

    @pl.when(s == 0)
    def _():
        pltpu.sync_copy(zrows_hbm.at[pl.ds(0, _ZTAIL)],
                        agg_sh.at[pl.ds(_NS * _ZR, _ZTAIL)])

    plsc.subcore_barrier()

    def body(j, carry):
        cp0 = pltpu.async_copy(h_hbm.at[src_v.at[2 * j]], buf_v.at[0], sem0)
        cp1 = pltpu.async_copy(h_hbm.at[src_v.at[2 * j + 1]], buf_v.at[1], sem1)
        cp0.wait()
        pltpu.sync_copy(buf_v.at[0], agg_sh.at[dst_v.at[2 * j]], add=True)
        cp1.wait()
        pltpu.sync_copy(buf_v.at[1], agg_sh.at[dst_v.at[2 * j + 1]], add=True)
        return carry

    lax.fori_loop(0, nlp_v[...][0], body, 0)
    plsc.subcore_barrier()
    pltpu.sync_copy(agg_sh.at[pl.ds(s * _ZR, _ZR)],
                    out_hbm.at[c, pl.ds(s * _ZR, _ZR)])

    @pl.when(s == 0)
    def _():
        pltpu.sync_copy(agg_sh.at[pl.ds(_NS * _ZR, _WTAIL)],
                        out_hbm.at[c, pl.ds(_NS * _ZR, _WTAIL)])


@functools.cache
def _sc_kernels():
    sc_mesh = plsc.VectorSubcoreMesh(
        core_axis_name="c", subcore_axis_name="s",
        num_cores=_NC, num_subcores=_NS)
    sc_prep = pl.kernel(
        _sc_prep_body,
        out_type=(
            jax.ShapeDtypeStruct((_NC, _NS, _NCH2, _CH), jnp.int32),
            jax.ShapeDtypeStruct((_NC, _NS, _NCH2, _CH), jnp.int32),
            jax.ShapeDtypeStruct((_NC, _NS, 16), jnp.int32),
        ),
        mesh=sc_mesh,
        scratch_types=[
            pltpu.VMEM((_HLF1, _CH), jnp.int32),
            pltpu.VMEM((_HLF1, _CH), jnp.int32),
            pltpu.VMEM((_NCH2, _CH), jnp.int32),
            pltpu.VMEM((_NCH2, _CH), jnp.int32),
            pltpu.VMEM((16,), jnp.int32),
        ],
        compiler_params=pltpu.CompilerParams(needs_layout_passes=False),
    )
    sc_agg = pl.kernel(
        _sc_agg_body,
        out_type=jax.ShapeDtypeStruct((_NC, _HALF, _D), jnp.float32),
        mesh=sc_mesh,
        scratch_types=[
            pltpu.VMEM((_NCH2, _CH), jnp.int32),
            pltpu.VMEM((_NCH2, _CH), jnp.int32),
            pltpu.VMEM((16,), jnp.int32),
            pltpu.VMEM((2, _CH, _D), jnp.float32),
            pltpu.SemaphoreType.DMA,
            pltpu.SemaphoreType.DMA,
            pltpu.VMEM_SHARED((_AROWS, _D), jnp.float32),
        ],
    )
    return sc_prep, sc_agg


def _tc_prep_body(batch_ref, cnt_ref, pnt_ref, deginv_ref):
    b = batch_ref[...]
    gids = lax.broadcasted_iota(jnp.int32, (_N, _G), 1)
    p = (b == gids).astype(jnp.float32)
    gcnt = jnp.sum(p, axis=0, keepdims=True)
    pnt_ref[...] = p / jnp.maximum(gcnt, 1.0)
    deg = jnp.concatenate([cnt_ref[0, :, :1], cnt_ref[1, :, :1]], axis=0)
    deginv_ref[...] = 1.0 / jnp.maximum(deg, 1.0)


_tc_prep = pl.pallas_call(
    _tc_prep_body,
    out_shape=(
        jax.ShapeDtypeStruct((_N, _G), jnp.float32),
        jax.ShapeDtypeStruct((_N, 1), jnp.float32),
    ),
)


def _tc_a_body(h_ref, parts_ref, deginv_ref, w1_ref, b1_ref, w2_ref, b2_ref,
               z_ref, stats_ref):
    i = pl.program_id(0)
    agg = parts_ref[0] * deginv_ref[...]
    zin = h_ref[...] + agg
    t = jnp.dot(zin, w1_ref[...], preferred_element_type=jnp.float32)
    t = jnp.maximum(t + b1_ref[...], 0.0)
    z = jnp.dot(t, w2_ref[...], preferred_element_type=jnp.float32)
    z = jnp.maximum(z + b2_ref[...], 0.0)
    z_ref[...] = z
    st = jnp.concatenate(
        [jnp.sum(z, axis=0, keepdims=True),
         jnp.sum(z * z, axis=0, keepdims=True),
         jnp.zeros((6, _D), jnp.float32)], axis=0)

    @pl.when(i == 0)
    def _():
        stats_ref[...] = st

    @pl.when(i != 0)
    def _():
        stats_ref[...] += st


_tc_a = pl.pallas_call(
    _tc_a_body,
    grid=(_NBLK,),
    in_specs=[
        pl.BlockSpec((_BLK, _D), lambda i: (i, 0)),
        pl.BlockSpec((1, _BLK, _D),
                     lambda i: (i // (_NBLK // _NC), i % (_NBLK // _NC), 0)),
        pl.BlockSpec((_BLK, 1), lambda i: (i, 0)),
        pl.BlockSpec((_D, _D), lambda i: (0, 0)),
        pl.BlockSpec((1, _D), lambda i: (0, 0)),
        pl.BlockSpec((_D, _D), lambda i: (0, 0)),
        pl.BlockSpec((1, _D), lambda i: (0, 0)),
    ],
    out_specs=(
        pl.BlockSpec((_BLK, _D), lambda i: (i, 0)),
        pl.BlockSpec((8, _D), lambda i: (0, 0)),
    ),
    out_shape=(
        jax.ShapeDtypeStruct((_N, _D), jnp.float32),
        jax.ShapeDtypeStruct((8, _D), jnp.float32),
    ),
)


def _tc_b_body(z_ref, stats_ref, g_ref, be_ref, npin_ref, pnt_ref, gpin_ref,
               h_ref, npout_ref, gpout_ref):
    i = pl.program_id(0)
    mu = stats_ref[0:1, :] / _N
    var = stats_ref[1:2, :] / _N - mu * mu
    inv = lax.rsqrt(var + 1e-5)
    hb = (z_ref[...] - mu) * (inv * g_ref[...]) + be_ref[...]
    h_ref[...] = hb
    npout_ref[...] = npin_ref[...] + hb
    contrib = lax.dot_general(pnt_ref[...], hb, (((0,), (0,)), ((), ())),
                              preferred_element_type=jnp.float32)

    @pl.when(i == 0)
    def _():
        gpout_ref[...] = gpin_ref[...] + contrib

    @pl.when(i != 0)
    def _():
        gpout_ref[...] += contrib


_tc_b = pl.pallas_call(
    _tc_b_body,
    grid=(_NBLK,),
    in_specs=[
        pl.BlockSpec((_BLK, _D), lambda i: (i, 0)),
        pl.BlockSpec((8, _D), lambda i: (0, 0)),
        pl.BlockSpec((1, _D), lambda i: (0, 0)),
        pl.BlockSpec((1, _D), lambda i: (0, 0)),
        pl.BlockSpec((_BLK, _D), lambda i: (i, 0)),
        pl.BlockSpec((_BLK, _G), lambda i: (i, 0)),
        pl.BlockSpec((_G, _D), lambda i: (0, 0)),
    ],
    out_specs=(
        pl.BlockSpec((_BLK, _D), lambda i: (i, 0)),
        pl.BlockSpec((_BLK, _D), lambda i: (i, 0)),
        pl.BlockSpec((_G, _D), lambda i: (0, 0)),
    ),
    out_shape=(
        jax.ShapeDtypeStruct((_N, _D), jnp.float32),
        jax.ShapeDtypeStruct((_N, _D), jnp.float32),
        jax.ShapeDtypeStruct((_G, _D), jnp.float32),
    ),
)


def kernel(x, params, edge_index, batch):
    src20 = edge_index[0].reshape(_NS, _NCH, _CH)
    dst20 = edge_index[1].reshape(_NS, _NCH, _CH)
    zrows = jnp.zeros((_ZR, _D), jnp.float32)
    ones_nd = jnp.ones((_N, _D), jnp.float32)
    batch2 = batch.reshape(_N, 1)

    sc_prep, sc_agg = _sc_kernels()
    csrc, cdst, nlp = sc_prep(src20, dst20)
    cnt = sc_agg(ones_nd, csrc, cdst, nlp, zrows)
    pnt, deginv = _tc_prep(batch2, cnt)

    h = x
    npool = jnp.zeros((_N, _D), jnp.float32)
    gpool = jnp.zeros((_G, _D), jnp.float32)
    for l in range(_LAYERS):
        parts = sc_agg(h, csrc, cdst, nlp, zrows)
        z, stats = _tc_a(h, parts, deginv,
                         params['W1_%d' % l], params['b1_%d' % l].reshape(1, _D),
                         params['W2_%d' % l], params['b2_%d' % l].reshape(1, _D))
        h, npool, gpool = _tc_b(z, stats,
                                params['g_%d' % l].reshape(1, _D),
                                params['be_%d' % l].reshape(1, _D),
                                npool, pnt, gpool)
    return (npool, gpool)


# speedup vs baseline: 4.6183x; 1.0325x over previous
import functools

import jax
import jax.numpy as jnp
from jax import lax
from jax.experimental import pallas as pl
from jax.experimental.pallas import tpu as pltpu
from jax.experimental.pallas import tpu_sc as plsc

_N = 10000
_E = 320000
_D = 128
_G = 128
_LAYERS = 4

_NC = 2
_NS = 16
_NW = _NC * _NS

_EPT = _E // _NS
_CH = 80
_NCH = _EPT // _CH
_NCH2 = _NCH
_CAP = _NCH2 * _CH
_HLF1 = 128
_HALF = _N // _NC
_NDUMP = 64
_AROWS = _HALF + _NDUMP
_ZR = 312
_ZTAIL = _AROWS - _NS * _ZR
_WTAIL = _HALF - _NS * _ZR

_BLK = 1000
_NBLK = _N // _BLK


def _sc_prep_body(src20_hbm, dst20_hbm, csrc_hbm, cdst_hbm, nlp_hbm,
                  src_v, dst_v, csrc_v, cdst_v, nlp_v):
    c = lax.axis_index("c")
    s = lax.axis_index("s")
    base = c * _HALF

    nlp_v[...] = jnp.zeros((16,), jnp.int32)

    def gbody(j, carry):
        for k in range(_CH // 16):
            d = dst_v[j, pl.ds(k * 16, 16)]
            sv = src_v[j, pl.ds(k * 16, 16)]
            cntv = nlp_v[...]
            idx = d - base
            ok = (idx >= 0) & (idx < _HALF)
            cs = plsc.cumsum(jnp.where(ok, 1, 0))
            pos = jnp.maximum(cs + cntv - 1, 0)
            row = lax.div(pos, jnp.full((16,), _CH, jnp.int32))
            col = pos - row * _CH
            plsc.store_scatter(csrc_v, [row, col], sv, mask=ok)
            plsc.store_scatter(cdst_v, [row, col], idx, mask=ok)
            nlp_v[...] = cntv + plsc.all_reduce_population_count(ok)
        return carry

    for off, n in ((0, _HLF1), (_HLF1, _NCH - _HLF1)):
        pltpu.sync_copy(src20_hbm.at[s, pl.ds(off, n)], src_v.at[pl.ds(0, n)])
        pltpu.sync_copy(dst20_hbm.at[s, pl.ds(off, n)], dst_v.at[pl.ds(0, n)])
        lax.fori_loop(0, n, gbody, 0)
    cntv = nlp_v[...]

    lanes = lax.iota(jnp.int32, 16)
    zsrc = jnp.zeros((16,), jnp.int32)
    chv = jnp.full((16,), _CH, jnp.int32)
    nloopsv = lax.div(cntv + 2 * _CH - 1, jnp.full((16,), 2 * _CH, jnp.int32))
    roundv = nloopsv * (2 * _CH)
    for p in range(10):
        pos = cntv + p * 16 + lanes
        okp = pos < roundv
        row = lax.div(jnp.minimum(pos, _CAP - 1), chv)
        col = jnp.minimum(pos, _CAP - 1) - row * _CH
        plsc.store_scatter(csrc_v, [row, col], zsrc, mask=okp)
        plsc.store_scatter(cdst_v, [row, col],
                           _HALF + (pos & (_NDUMP - 1)), mask=okp)

    nlp_v[...] = nloopsv
    pltpu.sync_copy(csrc_v, csrc_hbm.at[c, s])
    pltpu.sync_copy(cdst_v, cdst_hbm.at[c, s])
    pltpu.sync_copy(nlp_v, nlp_hbm.at[c, s])


def _sc_agg_body(h_hbm, csrc_hbm, cdst_hbm, nlp_hbm, zrows_hbm, out_hbm,
                 src_v, dst_v, nlp_v, buf_v, g0, g1, s0, s1, agg_sh):
    c = lax.axis_index("c")
    s = lax.axis_index("s")
    pltpu.sync_copy(csrc_hbm.at[c, s], src_v)
    pltpu.sync_copy(cdst_hbm.at[c, s], dst_v)
    pltpu.sync_copy(nlp_hbm.at[c, s], nlp_v)
    pltpu.sync_copy(zrows_hbm, agg_sh.at[pl.ds(s * _ZR, _ZR)])

    @pl.when(s == 0)
    def _():
        pltpu.sync_copy(zrows_hbm.at[pl.ds(0, _ZTAIL)],
                        agg_sh.at[pl.ds(_NS * _ZR, _ZTAIL)])

    plsc.subcore_barrier()
    nlp = nlp_v[...][0]

    @pl.when(nlp > 0)
    def _():
        pltpu.async_copy(h_hbm.at[src_v.at[0]], buf_v.at[0], g0)
        pltpu.async_copy(h_hbm.at[src_v.at[1]], buf_v.at[1], g1)

    def body(j, carry):
        pltpu.make_async_copy(
            h_hbm.at[src_v.at[2 * j]], buf_v.at[0], g0).wait()
        pltpu.async_copy(
            buf_v.at[0], agg_sh.at[dst_v.at[2 * j]], s0, add=True)
        pltpu.make_async_copy(
            h_hbm.at[src_v.at[2 * j + 1]], buf_v.at[1], g1).wait()
        pltpu.async_copy(
            buf_v.at[1], agg_sh.at[dst_v.at[2 * j + 1]], s1, add=True)

        @pl.when(j + 1 < nlp)
        def _():
            pltpu.make_async_copy(
                buf_v.at[0], agg_sh.at[dst_v.at[2 * j]], s0).wait()
            pltpu.async_copy(
                h_hbm.at[src_v.at[2 * j + 2]], buf_v.at[0], g0)
            pltpu.make_async_copy(
                buf_v.at[1], agg_sh.at[dst_v.at[2 * j + 1]], s1).wait()
            pltpu.async_copy(
                h_hbm.at[src_v.at[2 * j + 3]], buf_v.at[1], g1)
        return carry

    lax.fori_loop(0, nlp, body, 0)

    @pl.when(nlp > 0)
    def _():
        pltpu.make_async_copy(
            buf_v.at[0], agg_sh.at[dst_v.at[2 * nlp - 2]], s0).wait()
        pltpu.make_async_copy(
            buf_v.at[1], agg_sh.at[dst_v.at[2 * nlp - 1]], s1).wait()

    plsc.subcore_barrier()
    pltpu.sync_copy(agg_sh.at[pl.ds(s * _ZR, _ZR)],
                    out_hbm.at[c, pl.ds(s * _ZR, _ZR)])

    @pl.when(s == 0)
    def _():
        pltpu.sync_copy(agg_sh.at[pl.ds(_NS * _ZR, _WTAIL)],
                        out_hbm.at[c, pl.ds(_NS * _ZR, _WTAIL)])


@functools.cache
def _sc_kernels():
    sc_mesh = plsc.VectorSubcoreMesh(
        core_axis_name="c", subcore_axis_name="s",
        num_cores=_NC, num_subcores=_NS)
    sc_prep = pl.kernel(
        _sc_prep_body,
        out_type=(
            jax.ShapeDtypeStruct((_NC, _NS, _NCH2, _CH), jnp.int32),
            jax.ShapeDtypeStruct((_NC, _NS, _NCH2, _CH), jnp.int32),
            jax.ShapeDtypeStruct((_NC, _NS, 16), jnp.int32),
        ),
        mesh=sc_mesh,
        scratch_types=[
            pltpu.VMEM((_HLF1, _CH), jnp.int32),
            pltpu.VMEM((_HLF1, _CH), jnp.int32),
            pltpu.VMEM((_NCH2, _CH), jnp.int32),
            pltpu.VMEM((_NCH2, _CH), jnp.int32),
            pltpu.VMEM((16,), jnp.int32),
        ],
        compiler_params=pltpu.CompilerParams(needs_layout_passes=False),
    )
    sc_agg = pl.kernel(
        _sc_agg_body,
        out_type=jax.ShapeDtypeStruct((_NC, _HALF, _D), jnp.float32),
        mesh=sc_mesh,
        scratch_types=[
            pltpu.VMEM((_NCH2, _CH), jnp.int32),
            pltpu.VMEM((_NCH2, _CH), jnp.int32),
            pltpu.VMEM((16,), jnp.int32),
            pltpu.VMEM((2, _CH, _D), jnp.float32),
            pltpu.SemaphoreType.DMA,
            pltpu.SemaphoreType.DMA,
            pltpu.SemaphoreType.DMA,
            pltpu.SemaphoreType.DMA,
            pltpu.VMEM_SHARED((_AROWS, _D), jnp.float32),
        ],
    )
    return sc_prep, sc_agg


def _tc_prep_body(batch_ref, cnt_ref, pnt_ref, deginv_ref):
    b = batch_ref[...]
    gids = lax.broadcasted_iota(jnp.int32, (_N, _G), 1)
    p = (b == gids).astype(jnp.float32)
    gcnt = jnp.sum(p, axis=0, keepdims=True)
    pnt_ref[...] = p / jnp.maximum(gcnt, 1.0)
    deg = jnp.concatenate([cnt_ref[0, :, :1], cnt_ref[1, :, :1]], axis=0)
    deginv_ref[...] = 1.0 / jnp.maximum(deg, 1.0)


_tc_prep = pl.pallas_call(
    _tc_prep_body,
    out_shape=(
        jax.ShapeDtypeStruct((_N, _G), jnp.float32),
        jax.ShapeDtypeStruct((_N, 1), jnp.float32),
    ),
)


def _tc_a_body(h_ref, parts_ref, deginv_ref, w1_ref, b1_ref, w2_ref, b2_ref,
               z_ref, stats_ref):
    i = pl.program_id(0)
    agg = parts_ref[0] * deginv_ref[...]
    zin = h_ref[...] + agg
    t = jnp.dot(zin, w1_ref[...], preferred_element_type=jnp.float32)
    t = jnp.maximum(t + b1_ref[...], 0.0)
    z = jnp.dot(t, w2_ref[...], preferred_element_type=jnp.float32)
    z = jnp.maximum(z + b2_ref[...], 0.0)
    z_ref[...] = z
    st = jnp.concatenate(
        [jnp.sum(z, axis=0, keepdims=True),
         jnp.sum(z * z, axis=0, keepdims=True),
         jnp.zeros((6, _D), jnp.float32)], axis=0)

    @pl.when(i == 0)
    def _():
        stats_ref[...] = st

    @pl.when(i != 0)
    def _():
        stats_ref[...] += st


_tc_a = pl.pallas_call(
    _tc_a_body,
    grid=(_NBLK,),
    in_specs=[
        pl.BlockSpec((_BLK, _D), lambda i: (i, 0)),
        pl.BlockSpec((1, _BLK, _D),
                     lambda i: (i // (_NBLK // _NC), i % (_NBLK // _NC), 0)),
        pl.BlockSpec((_BLK, 1), lambda i: (i, 0)),
        pl.BlockSpec((_D, _D), lambda i: (0, 0)),
        pl.BlockSpec((1, _D), lambda i: (0, 0)),
        pl.BlockSpec((_D, _D), lambda i: (0, 0)),
        pl.BlockSpec((1, _D), lambda i: (0, 0)),
    ],
    out_specs=(
        pl.BlockSpec((_BLK, _D), lambda i: (i, 0)),
        pl.BlockSpec((8, _D), lambda i: (0, 0)),
    ),
    out_shape=(
        jax.ShapeDtypeStruct((_N, _D), jnp.float32),
        jax.ShapeDtypeStruct((8, _D), jnp.float32),
    ),
)


def _tc_b_body(z_ref, stats_ref, g_ref, be_ref, npin_ref, pnt_ref, gpin_ref,
               h_ref, npout_ref, gpout_ref):
    i = pl.program_id(0)
    mu = stats_ref[0:1, :] / _N
    var = stats_ref[1:2, :] / _N - mu * mu
    inv = lax.rsqrt(var + 1e-5)
    hb = (z_ref[...] - mu) * (inv * g_ref[...]) + be_ref[...]
    h_ref[...] = hb
    npout_ref[...] = npin_ref[...] + hb
    contrib = lax.dot_general(pnt_ref[...], hb, (((0,), (0,)), ((), ())),
                              preferred_element_type=jnp.float32)

    @pl.when(i == 0)
    def _():
        gpout_ref[...] = gpin_ref[...] + contrib

    @pl.when(i != 0)
    def _():
        gpout_ref[...] += contrib


_tc_b = pl.pallas_call(
    _tc_b_body,
    grid=(_NBLK,),
    in_specs=[
        pl.BlockSpec((_BLK, _D), lambda i: (i, 0)),
        pl.BlockSpec((8, _D), lambda i: (0, 0)),
        pl.BlockSpec((1, _D), lambda i: (0, 0)),
        pl.BlockSpec((1, _D), lambda i: (0, 0)),
        pl.BlockSpec((_BLK, _D), lambda i: (i, 0)),
        pl.BlockSpec((_BLK, _G), lambda i: (i, 0)),
        pl.BlockSpec((_G, _D), lambda i: (0, 0)),
    ],
    out_specs=(
        pl.BlockSpec((_BLK, _D), lambda i: (i, 0)),
        pl.BlockSpec((_BLK, _D), lambda i: (i, 0)),
        pl.BlockSpec((_G, _D), lambda i: (0, 0)),
    ),
    out_shape=(
        jax.ShapeDtypeStruct((_N, _D), jnp.float32),
        jax.ShapeDtypeStruct((_N, _D), jnp.float32),
        jax.ShapeDtypeStruct((_G, _D), jnp.float32),
    ),
)


def kernel(x, params, edge_index, batch):
    src20 = edge_index[0].reshape(_NS, _NCH, _CH)
    dst20 = edge_index[1].reshape(_NS, _NCH, _CH)
    zrows = jnp.zeros((_ZR, _D), jnp.float32)
    ones_nd = jnp.ones((_N, _D), jnp.float32)
    batch2 = batch.reshape(_N, 1)

    sc_prep, sc_agg = _sc_kernels()
    csrc, cdst, nlp = sc_prep(src20, dst20)
    cnt = sc_agg(ones_nd, csrc, cdst, nlp, zrows)
    pnt, deginv = _tc_prep(batch2, cnt)

    h = x
    npool = jnp.zeros((_N, _D), jnp.float32)
    gpool = jnp.zeros((_G, _D), jnp.float32)
    for l in range(_LAYERS):
        parts = sc_agg(h, csrc, cdst, nlp, zrows)
        z, stats = _tc_a(h, parts, deginv,
                         params['W1_%d' % l], params['b1_%d' % l].reshape(1, _D),
                         params['W2_%d' % l], params['b2_%d' % l].reshape(1, _D))
        h, npool, gpool = _tc_b(z, stats,
                                params['g_%d' % l].reshape(1, _D),
                                params['be_%d' % l].reshape(1, _D),
                                npool, pnt, gpool)
    return (npool, gpool)
